# baseline (speedup 1.0000x reference)
import functools

import jax
import jax.numpy as jnp
from jax import lax
from jax.experimental import pallas as pl
from jax.experimental.pallas import tpu as pltpu
from jax.experimental.pallas import tpu_sc as plsc

BS = 256
NJ = 17
ENC_HID = 512
TOKEN_NUM = 34
TOKEN_DIM = 64
K = 8192
DEC_HID = 32

ENC_BT = 32
VQ_ROWS = 272
N_LAYER_ARRS = 12


def _ln(x, g, b):
    m = jnp.mean(x, axis=-1, keepdims=True)
    xc = x - m
    v = jnp.mean(xc * xc, axis=-1, keepdims=True)
    return xc / jnp.sqrt(v + 1e-5) * g + b


def _gelu(x):
    return x * 0.5 * (1.0 + lax.erf(x * (2.0 ** -0.5)))


def _mixer_block(x, bt, hid, tok, lr):
    (g1, b1, tw1, tb1, tw2, tb2, g2, b2, cw1, cb1, cw2, cb2) = lr
    y = _ln(x, g1, b1)
    yt = jnp.swapaxes(y, 1, 2).reshape(bt * hid, tok)
    u = _gelu(yt @ tw1 + tb1) @ tw2 + tb2
    y2 = jnp.swapaxes(u.reshape(bt, hid, tok), 1, 2)
    z_in = _ln(x + y2, g2, b2)
    z2d = z_in.reshape(bt * tok, hid)
    z = (_gelu(z2d @ cw1 + cb1) @ cw2 + cb2).reshape(bt, tok, hid)
    return x + y2 + z


def _enc_body(*refs):
    (joints_ref, w_ref, inv_ref, sw_ref, sb_ref) = refs[:5]
    layer_refs = refs[5:5 + 4 * N_LAYER_ARRS]
    (eg_ref, eb_ref, tokw_ref, tokb_ref, fw_ref, fb_ref, o_ref) = refs[5 + 4 * N_LAYER_ARRS:]
    bt = ENC_BT
    x = (joints_ref[...].reshape(bt * NJ, 3) @ sw_ref[...] + sb_ref[...])
    x = x.reshape(bt, NJ, ENC_HID)
    w = w_ref[...]
    x = x * w + inv_ref[...] * (1.0 - w)
    for li in range(4):
        lr = [r[...] for r in layer_refs[li * N_LAYER_ARRS:(li + 1) * N_LAYER_ARRS]]
        x = _mixer_block(x, bt, ENC_HID, NJ, lr)
    e = _ln(x, eg_ref[...], eb_ref[...])
    et = jnp.swapaxes(e, 1, 2).reshape(bt * ENC_HID, NJ)
    t = (et @ tokw_ref[...] + tokb_ref[...]).reshape(bt, ENC_HID, TOKEN_NUM)
    t = jnp.swapaxes(t, 1, 2).reshape(bt * TOKEN_NUM, ENC_HID)
    ef = t @ fw_ref[...] + fb_ref[...]
    o_ref[...] = ef.reshape(bt, TOKEN_NUM, TOKEN_DIM)


def _vq_body(ef_ref, cbt_ref, idx_ref):
    ef = ef_ref[...]
    cbt = cbt_ref[...]
    mm = jnp.dot(ef, cbt, preferred_element_type=jnp.float32)
    ef_sq = jnp.sum(ef * ef, axis=1, keepdims=True)
    cb_sq = jnp.sum(cbt * cbt, axis=0, keepdims=True)
    d = (ef_sq + cb_sq) - 2.0 * mm
    mind = jnp.min(d, axis=1, keepdims=True)
    iota = lax.broadcasted_iota(jnp.int32, d.shape, 1)
    idx = jnp.min(jnp.where(d == mind, iota, jnp.int32(K)), axis=1)
    idx_ref[...] = idx.reshape(1, 1, VQ_ROWS)


def _dec_body(*refs):
    (part_ref, ef_ref, dtw_ref, dtb_ref, dsw_ref, dsb_ref) = refs[:6]
    layer_refs = refs[6:6 + N_LAYER_ARRS]
    (dg_ref, db_ref, rw_ref, rb_ref, rec_ref, loss_ref) = refs[6 + N_LAYER_ARRS:]
    part = part_ref[...][:, :, :TOKEN_DIM]
    ef = ef_ref[...]
    diff = part - ef
    loss_ref[...] = jnp.mean(diff * diff).reshape(1, 1)
    p = ef + diff
    pt = jnp.swapaxes(p, 1, 2).reshape(BS * TOKEN_DIM, TOKEN_NUM)
    pt = pt @ dtw_ref[...] + dtb_ref[...]
    p2 = jnp.swapaxes(pt.reshape(BS, TOKEN_DIM, NJ), 1, 2)
    df = (p2.reshape(BS * NJ, TOKEN_DIM) @ dsw_ref[...] + dsb_ref[...])
    df = df.reshape(BS, NJ, DEC_HID)
    lr = [r[...] for r in layer_refs]
    df = _mixer_block(df, BS, DEC_HID, NJ, lr)
    df = _ln(df, dg_ref[...], db_ref[...])
    rec = df.reshape(BS * NJ, DEC_HID) @ rw_ref[...] + rb_ref[...]
    rec_ref[...] = rec.reshape(BS, NJ, 3)


def _const_spec(shape):
    return pl.BlockSpec(shape, lambda i: (0,) * len(shape))


def _r2(a):
    return a.reshape(1, -1) if a.ndim == 1 else a


def _layer_arrs(p):
    return [_r2(p[k]) for k in ('ln1_g', 'ln1_b', 'tw1', 'tb1', 'tw2', 'tb2',
                                'ln2_g', 'ln2_b', 'cw1', 'cb1', 'cw2', 'cb2')]


def _sc_gather(codebook, idx):
    info = plsc.get_sparse_core_info()
    nc, ns = info.num_cores, info.num_subcores
    nw = nc * ns
    n = idx.shape[0]
    bpw = n // nw
    table = jnp.pad(codebook, ((0, 0), (0, 128 - TOKEN_DIM)))
    mesh = plsc.VectorSubcoreMesh(core_axis_name="c", subcore_axis_name="s")

    @functools.partial(
        pl.kernel, mesh=mesh,
        out_type=jax.ShapeDtypeStruct((n, 128), jnp.float32),
        scratch_types=[
            pltpu.VMEM((bpw,), jnp.int32),
            pltpu.VMEM((bpw, 128), jnp.float32),
            pltpu.SemaphoreType.DMA,
        ],
    )
    def gather_k(table_hbm, idx_hbm, out_hbm, idx_v, rows_v, sem):
        wid = lax.axis_index("s") * nc + lax.axis_index("c")
        base = wid * bpw
        pltpu.sync_copy(idx_hbm.at[pl.ds(base, bpw)], idx_v)
        pltpu.async_copy(table_hbm.at[idx_v], rows_v, sem).wait()
        pltpu.sync_copy(rows_v, out_hbm.at[pl.ds(base, bpw)])

    return gather_k(table, idx)


def kernel(joints, joints_feature, cls_logits, mask, params, codebook):
    w = mask[..., None].astype(jnp.float32)
    enc_in = [joints, w, params['invisible_token'],
              params['start_w'], _r2(params['start_b'])]
    for p in params['enc_layers']:
        enc_in += _layer_arrs(p)
    enc_in += [_r2(params['enc_ln_g']), _r2(params['enc_ln_b']),
               params['token_mlp_w'], _r2(params['token_mlp_b']),
               params['feat_w'], _r2(params['feat_b'])]
    enc_specs = [pl.BlockSpec((ENC_BT, NJ, 3), lambda i: (i, 0, 0)),
                 pl.BlockSpec((ENC_BT, NJ, 1), lambda i: (i, 0, 0))]
    enc_specs += [_const_spec(a.shape) for a in enc_in[2:]]
    ef3 = pl.pallas_call(
        _enc_body,
        grid=(BS // ENC_BT,),
        in_specs=enc_specs,
        out_specs=pl.BlockSpec((ENC_BT, TOKEN_NUM, TOKEN_DIM), lambda i: (i, 0, 0)),
        out_shape=jax.ShapeDtypeStruct((BS, TOKEN_NUM, TOKEN_DIM), jnp.float32),
    )(*enc_in)

    n_rows = BS * TOKEN_NUM
    ef2 = ef3.reshape(n_rows, TOKEN_DIM)
    cbt = codebook.T
    n_tiles = n_rows // VQ_ROWS
    idx3 = pl.pallas_call(
        _vq_body,
        grid=(n_tiles,),
        in_specs=[pl.BlockSpec((VQ_ROWS, TOKEN_DIM), lambda i: (i, 0)),
                  _const_spec((TOKEN_DIM, K))],
        out_specs=pl.BlockSpec((1, 1, VQ_ROWS), lambda i: (i, 0, 0)),
        out_shape=jax.ShapeDtypeStruct((n_tiles, 1, VQ_ROWS), jnp.int32),
    )(ef2, cbt)
    idx = idx3.reshape(n_rows)

    part = _sc_gather(codebook, idx)

    dec_in = [part.reshape(BS, TOKEN_NUM, 128), ef3,
              params['dec_tok_w'], _r2(params['dec_tok_b']),
              params['dec_start_w'], _r2(params['dec_start_b'])]
    dec_in += _layer_arrs(params['dec_layers'][0])
    dec_in += [_r2(params['dec_ln_g']), _r2(params['dec_ln_b']),
               params['rec_w'], _r2(params['rec_b'])]
    rec, loss = pl.pallas_call(
        _dec_body,
        out_shape=[jax.ShapeDtypeStruct((BS, NJ, 3), jnp.float32),
                   jax.ShapeDtypeStruct((1, 1), jnp.float32)],
    )(*dec_in)
    return rec, idx, loss.reshape(())


# speedup vs baseline: 1.0008x; 1.0008x over previous
import functools

import jax
import jax.numpy as jnp
from jax import lax
from jax.experimental import pallas as pl
from jax.experimental.pallas import tpu as pltpu
from jax.experimental.pallas import tpu_sc as plsc

BS = 256
NJ = 17
ENC_HID = 512
TOKEN_NUM = 34
TOKEN_DIM = 64
K = 8192
DEC_HID = 32

ENC_BT = 32
VQ_ROWS = 272
N_LAYER_ARRS = 12


def _ln(x, g, b):
    m = jnp.mean(x, axis=-1, keepdims=True)
    xc = x - m
    v = jnp.mean(xc * xc, axis=-1, keepdims=True)
    return xc / jnp.sqrt(v + 1e-5) * g + b


def _gelu(x):
    return x * 0.5 * (1.0 + lax.erf(x * (2.0 ** -0.5)))


def _mixer_block(x, bt, hid, tok, lr):
    (g1, b1, tw1, tb1, tw2, tb2, g2, b2, cw1, cb1, cw2, cb2) = lr
    y = _ln(x, g1, b1)
    yt = jnp.swapaxes(y, 1, 2).reshape(bt * hid, tok)
    u = _gelu(yt @ tw1 + tb1) @ tw2 + tb2
    y2 = jnp.swapaxes(u.reshape(bt, hid, tok), 1, 2)
    z_in = _ln(x + y2, g2, b2)
    z2d = z_in.reshape(bt * tok, hid)
    z = (_gelu(z2d @ cw1 + cb1) @ cw2 + cb2).reshape(bt, tok, hid)
    return x + y2 + z


def _enc_body(*refs):
    (joints_ref, w_ref, inv_ref, sw_ref, sb_ref) = refs[:5]
    layer_refs = refs[5:5 + 4 * N_LAYER_ARRS]
    (eg_ref, eb_ref, tokw_ref, tokb_ref, fw_ref, fb_ref, o_ref) = refs[5 + 4 * N_LAYER_ARRS:]
    bt = ENC_BT
    x = (joints_ref[...].reshape(bt * NJ, 3) @ sw_ref[...] + sb_ref[...])
    x = x.reshape(bt, NJ, ENC_HID)
    w = w_ref[...]
    x = x * w + inv_ref[...] * (1.0 - w)
    for li in range(4):
        lr = [r[...] for r in layer_refs[li * N_LAYER_ARRS:(li + 1) * N_LAYER_ARRS]]
        x = _mixer_block(x, bt, ENC_HID, NJ, lr)
    e = _ln(x, eg_ref[...], eb_ref[...])
    et = jnp.swapaxes(e, 1, 2).reshape(bt * ENC_HID, NJ)
    t = (et @ tokw_ref[...] + tokb_ref[...]).reshape(bt, ENC_HID, TOKEN_NUM)
    t = jnp.swapaxes(t, 1, 2).reshape(bt * TOKEN_NUM, ENC_HID)
    ef = t @ fw_ref[...] + fb_ref[...]
    o_ref[...] = ef.reshape(bt, TOKEN_NUM, TOKEN_DIM)


def _vq_body(ef_ref, cbt_ref, idx_ref):
    ef = ef_ref[...]
    cbt = cbt_ref[...]
    mm = jnp.dot(ef, cbt, preferred_element_type=jnp.float32)
    ef_sq = jnp.sum(ef * ef, axis=1, keepdims=True)
    cb_sq = jnp.sum(cbt * cbt, axis=0, keepdims=True)
    d = (ef_sq + cb_sq) - 2.0 * mm
    mind = jnp.min(d, axis=1, keepdims=True)
    iota = lax.broadcasted_iota(jnp.int32, d.shape, 1)
    idx = jnp.min(jnp.where(d == mind, iota, jnp.int32(K)), axis=1)
    idx_ref[...] = idx.reshape(1, 1, VQ_ROWS)


def _dec_body(*refs):
    (part_ref, ef_ref, dtw_ref, dtb_ref, dsw_ref, dsb_ref) = refs[:6]
    layer_refs = refs[6:6 + N_LAYER_ARRS]
    (dg_ref, db_ref, rw_ref, rb_ref, rec_ref, loss_ref) = refs[6 + N_LAYER_ARRS:]
    part = part_ref[...][:, :, :TOKEN_DIM]
    ef = ef_ref[...]
    diff = part - ef
    loss_ref[...] = jnp.mean(diff * diff).reshape(1, 1)
    p = ef + diff
    pt = jnp.swapaxes(p, 1, 2).reshape(BS * TOKEN_DIM, TOKEN_NUM)
    pt = pt @ dtw_ref[...] + dtb_ref[...]
    p2 = jnp.swapaxes(pt.reshape(BS, TOKEN_DIM, NJ), 1, 2)
    df = (p2.reshape(BS * NJ, TOKEN_DIM) @ dsw_ref[...] + dsb_ref[...])
    df = df.reshape(BS, NJ, DEC_HID)
    lr = [r[...] for r in layer_refs]
    df = _mixer_block(df, BS, DEC_HID, NJ, lr)
    df = _ln(df, dg_ref[...], db_ref[...])
    rec = df.reshape(BS * NJ, DEC_HID) @ rw_ref[...] + rb_ref[...]
    rec_ref[...] = rec.reshape(BS, NJ, 3)


def _const_spec(shape):
    return pl.BlockSpec(shape, lambda i: (0,) * len(shape))


def _r2(a):
    return a.reshape(1, -1) if a.ndim == 1 else a


def _layer_arrs(p):
    return [_r2(p[k]) for k in ('ln1_g', 'ln1_b', 'tw1', 'tb1', 'tw2', 'tb2',
                                'ln2_g', 'ln2_b', 'cw1', 'cb1', 'cw2', 'cb2')]


def _sc_gather(codebook, idx):
    info = plsc.get_sparse_core_info()
    nc, ns = info.num_cores, info.num_subcores
    nw = nc * ns
    n = idx.shape[0]
    bpw = n // nw
    table = jnp.pad(codebook, ((0, 0), (0, 128 - TOKEN_DIM)))
    mesh = plsc.VectorSubcoreMesh(core_axis_name="c", subcore_axis_name="s")

    @functools.partial(
        pl.kernel, mesh=mesh,
        out_type=jax.ShapeDtypeStruct((n, 128), jnp.float32),
        scratch_types=[
            pltpu.VMEM((bpw,), jnp.int32),
            pltpu.VMEM((bpw, 128), jnp.float32),
            pltpu.SemaphoreType.DMA,
        ],
    )
    def gather_k(table_hbm, idx_hbm, out_hbm, idx_v, rows_v, sem):
        wid = lax.axis_index("s") * nc + lax.axis_index("c")
        base = wid * bpw
        pltpu.sync_copy(idx_hbm.at[pl.ds(base, bpw)], idx_v)
        nb = bpw // 16
        cps = [pltpu.async_copy(table_hbm.at[idx_v[pl.ds(j * 16, 16)]],
                                rows_v.at[pl.ds(j * 16, 16)], sem)
               for j in range(nb)]
        for cp in cps:
            cp.wait()
        pltpu.sync_copy(rows_v, out_hbm.at[pl.ds(base, bpw)])

    return gather_k(table, idx)


def kernel(joints, joints_feature, cls_logits, mask, params, codebook):
    w = mask[..., None].astype(jnp.float32)
    enc_in = [joints, w, params['invisible_token'],
              params['start_w'], _r2(params['start_b'])]
    for p in params['enc_layers']:
        enc_in += _layer_arrs(p)
    enc_in += [_r2(params['enc_ln_g']), _r2(params['enc_ln_b']),
               params['token_mlp_w'], _r2(params['token_mlp_b']),
               params['feat_w'], _r2(params['feat_b'])]
    enc_specs = [pl.BlockSpec((ENC_BT, NJ, 3), lambda i: (i, 0, 0)),
                 pl.BlockSpec((ENC_BT, NJ, 1), lambda i: (i, 0, 0))]
    enc_specs += [_const_spec(a.shape) for a in enc_in[2:]]
    ef3 = pl.pallas_call(
        _enc_body,
        grid=(BS // ENC_BT,),
        in_specs=enc_specs,
        out_specs=pl.BlockSpec((ENC_BT, TOKEN_NUM, TOKEN_DIM), lambda i: (i, 0, 0)),
        out_shape=jax.ShapeDtypeStruct((BS, TOKEN_NUM, TOKEN_DIM), jnp.float32),
    )(*enc_in)

    n_rows = BS * TOKEN_NUM
    ef2 = ef3.reshape(n_rows, TOKEN_DIM)
    cbt = codebook.T
    n_tiles = n_rows // VQ_ROWS
    idx3 = pl.pallas_call(
        _vq_body,
        grid=(n_tiles,),
        in_specs=[pl.BlockSpec((VQ_ROWS, TOKEN_DIM), lambda i: (i, 0)),
                  _const_spec((TOKEN_DIM, K))],
        out_specs=pl.BlockSpec((1, 1, VQ_ROWS), lambda i: (i, 0, 0)),
        out_shape=jax.ShapeDtypeStruct((n_tiles, 1, VQ_ROWS), jnp.int32),
    )(ef2, cbt)
    idx = idx3.reshape(n_rows)

    part = _sc_gather(codebook, idx)

    dec_in = [part.reshape(BS, TOKEN_NUM, 128), ef3,
              params['dec_tok_w'], _r2(params['dec_tok_b']),
              params['dec_start_w'], _r2(params['dec_start_b'])]
    dec_in += _layer_arrs(params['dec_layers'][0])
    dec_in += [_r2(params['dec_ln_g']), _r2(params['dec_ln_b']),
               params['rec_w'], _r2(params['rec_b'])]
    rec, loss = pl.pallas_call(
        _dec_body,
        out_shape=[jax.ShapeDtypeStruct((BS, NJ, 3), jnp.float32),
                   jax.ShapeDtypeStruct((1, 1), jnp.float32)],
    )(*dec_in)
    return rec, idx, loss.reshape(())
